# Initial kernel scaffold; baseline (speedup 1.0000x reference)
import jax
import jax.numpy as jnp
from jax.experimental import pallas as pl


def kernel(h, edge_index, msg_W, msg_b, msg_rev_W, msg_rev_b, W_ih, b_ih, W_hh, b_hh):
    raise NotImplementedError("write your pallas kernel here")


# speedup vs baseline: 3.8729x; 3.8729x over previous
import functools

import jax
import jax.numpy as jnp
from jax import lax
from jax.experimental import pallas as pl
from jax.experimental.pallas import tpu as pltpu
from jax.experimental.pallas import tpu_sc as plsc

_N = 10000
_E = 160000
_D = 256
_L = 3
_CH = 128
_NCH = 2 * _D // _CH
_NS = 16
_NC = 2
_EPT = _E // _NS
_K = 80
_ITERS = _EPT // _K
_RPT_PAD = 640
_FPT = 624
_ROWBLK = 256
_GRID_R = (_N + _ROWBLK - 1) // _ROWBLK


def _deg_body(ei0_hbm, ei1_hbm, ones_hbm, zeros_hbm, out_hbm,
              idx_v, ones_v, dacc, ):
    c = lax.axis_index("c")
    s = lax.axis_index("s")
    pltpu.sync_copy(ones_hbm, ones_v)
    for core in range(_NC):
        @pl.when(c == core)
        def _(core=core):
            src = ei1_hbm if core == 0 else ei0_hbm
            pltpu.sync_copy(zeros_hbm, dacc.at[pl.ds(s * _RPT_PAD, _RPT_PAD)])
            plsc.subcore_barrier()

            def body(i, carry):
                base = pl.multiple_of(s * _EPT + i * _K, 8)
                pltpu.sync_copy(src.at[pl.ds(base, _K)], idx_v)
                pltpu.sync_copy(ones_v, dacc.at[idx_v], add=True)
                return carry

            lax.fori_loop(0, _ITERS, body, 0)
            plsc.subcore_barrier()
            pltpu.sync_copy(dacc.at[pl.ds(s * _FPT, _FPT)],
                            out_hbm.at[core].at[pl.ds(s * _FPT, _FPT)])

            @pl.when(s == _NS - 1)
            def _():
                tail = _NS * _FPT
                pltpu.sync_copy(dacc.at[pl.ds(tail, _N - tail)],
                                out_hbm.at[core].at[pl.ds(tail, _N - tail)])


@functools.cache
def _deg_kernel():
    return pl.kernel(
        _deg_body,
        out_type=jax.ShapeDtypeStruct((2, _N, 16), jnp.float32),
        mesh=plsc.VectorSubcoreMesh(core_axis_name="c", subcore_axis_name="s",
                                    num_cores=_NC, num_subcores=_NS),
        scratch_types=[
            pltpu.VMEM((_K,), jnp.int32),
            pltpu.VMEM((_K, 16), jnp.float32),
            pltpu.VMEM_SHARED((_NS * _RPT_PAD, 16), jnp.float32),
        ],
    )


def _scat_body(tabs_hbm, ei0_hbm, ei1_hbm, zeros_hbm, out_hbm,
               idx0_v, idx1_v, rows_v, acc, sem):
    c = lax.axis_index("c")
    s = lax.axis_index("s")
    for chunk in range(_NCH):
        @pl.when(c == chunk // 2)
        def _(chunk=chunk):
            pltpu.sync_copy(zeros_hbm, acc.at[pl.ds(s * _RPT_PAD, _RPT_PAD)])
            plsc.subcore_barrier()

            def body(i, carry):
                base = pl.multiple_of(s * _EPT + i * _K, 8)
                pltpu.sync_copy(ei0_hbm.at[pl.ds(base, _K)], idx0_v)
                pltpu.sync_copy(ei1_hbm.at[pl.ds(base, _K)], idx1_v)
                pltpu.async_copy(tabs_hbm.at[chunk].at[idx0_v], rows_v,
                                 sem).wait()
                pltpu.sync_copy(rows_v, acc.at[idx1_v], add=True)
                pltpu.async_copy(tabs_hbm.at[chunk + _NCH].at[idx1_v], rows_v,
                                 sem).wait()
                pltpu.sync_copy(rows_v, acc.at[idx0_v], add=True)
                return carry

            lax.fori_loop(0, _ITERS, body, 0)
            plsc.subcore_barrier()
            pltpu.sync_copy(acc.at[pl.ds(s * _FPT, _FPT)],
                            out_hbm.at[chunk].at[pl.ds(s * _FPT, _FPT)])

            @pl.when(s == _NS - 1)
            def _():
                tail = _NS * _FPT
                pltpu.sync_copy(acc.at[pl.ds(tail, _N - tail)],
                                out_hbm.at[chunk].at[pl.ds(tail, _N - tail)])

            plsc.subcore_barrier()


@functools.cache
def _scat_kernel():
    return pl.kernel(
        _scat_body,
        out_type=jax.ShapeDtypeStruct((_NCH, _N, _CH), jnp.float32),
        mesh=plsc.VectorSubcoreMesh(core_axis_name="c", subcore_axis_name="s",
                                    num_cores=_NC, num_subcores=_NS),
        scratch_types=[
            pltpu.VMEM((_K,), jnp.int32),
            pltpu.VMEM((_K,), jnp.int32),
            pltpu.VMEM((_K, _CH), jnp.float32),
            pltpu.VMEM_SHARED((_NS * _RPT_PAD, _CH), jnp.float32),
            pltpu.SemaphoreType.DMA,
        ],
    )


def _proj_body(h_ref, w_ref, out_ref):
    out_ref[0] = jnp.dot(h_ref[...], w_ref[0],
                         preferred_element_type=jnp.float32)


def _proj_tables(h, w_all):
    return pl.pallas_call(
        _proj_body,
        grid=(2 * _NCH, _GRID_R),
        in_specs=[
            pl.BlockSpec((_ROWBLK, _D), lambda g, i: (i, 0)),
            pl.BlockSpec((1, _D, _CH), lambda g, i: (g, 0, 0)),
        ],
        out_specs=pl.BlockSpec((1, _ROWBLK, _CH), lambda g, i: (g, i, 0)),
        out_shape=jax.ShapeDtypeStruct((2 * _NCH, _N, _CH), jnp.float32),
    )(h, w_all)


def _upd_body(s_ref, h_ref, di_ref, do_ref, w1d_ref, w2d_ref, b1_ref, b2_ref,
              wih_ref, bih_ref, whh_ref, bhh_ref, out_ref):
    f32 = jnp.float32
    h_blk = h_ref[...]
    di = di_ref[:, 0:1]
    do = do_ref[:, 0:1]
    gi = bih_ref[0] + jnp.zeros((_ROWBLK, 3 * _D), f32)
    for cidx in range(_NCH):
        p1d = jnp.dot(h_blk, w1d_ref[cidx], preferred_element_type=f32)
        p2d = jnp.dot(h_blk, w2d_ref[cidx], preferred_element_type=f32)
        aggr_c = (s_ref[cidx]
                  + di * (p1d + b1_ref[cidx])
                  + do * (p2d + b2_ref[cidx]))
        gi = gi + jnp.dot(aggr_c, wih_ref[cidx], preferred_element_type=f32)
    gh = jnp.dot(h_blk, whh_ref[...], preferred_element_type=f32) + bhh_ref[0]
    r = jax.nn.sigmoid(gi[:, :_D] + gh[:, :_D])
    z = jax.nn.sigmoid(gi[:, _D:2 * _D] + gh[:, _D:2 * _D])
    ng = jnp.tanh(gi[:, 2 * _D:] + r * gh[:, 2 * _D:])
    out_ref[...] = (1.0 - z) * ng + z * h_blk


def _gru_update(s_chunks, h, deg_in, deg_out, w1d, w2d, b1c, b2c,
                wih_t, bih, whh_t, bhh):
    full = lambda shape: pl.BlockSpec(shape, lambda i: (0,) * len(shape))
    return pl.pallas_call(
        _upd_body,
        grid=(_GRID_R,),
        in_specs=[
            pl.BlockSpec((_NCH, _ROWBLK, _CH), lambda i: (0, i, 0)),
            pl.BlockSpec((_ROWBLK, _D), lambda i: (i, 0)),
            pl.BlockSpec((_ROWBLK, 16), lambda i: (i, 0)),
            pl.BlockSpec((_ROWBLK, 16), lambda i: (i, 0)),
            full((_NCH, _D, _CH)),
            full((_NCH, _D, _CH)),
            full((_NCH, _CH)),
            full((_NCH, _CH)),
            full((_NCH, _CH, 3 * _D)),
            full((1, 3 * _D)),
            full((_D, 3 * _D)),
            full((1, 3 * _D)),
        ],
        out_specs=pl.BlockSpec((_ROWBLK, _D), lambda i: (i, 0)),
        out_shape=jax.ShapeDtypeStruct((_N, _D), jnp.float32),
    )(s_chunks, h, deg_in, deg_out, w1d, w2d, b1c, b2c,
      wih_t, bih, whh_t, bhh)


def kernel(h, edge_index, msg_W, msg_b, msg_rev_W, msg_rev_b,
           W_ih, b_ih, W_hh, b_hh):
    d = _D
    ei0 = edge_index[0].astype(jnp.int32)
    ei1 = edge_index[1].astype(jnp.int32)

    zeros16 = jnp.zeros((_RPT_PAD, 16), jnp.float32)
    ones16 = jnp.ones((_K, 16), jnp.float32)
    zeros_ch = jnp.zeros((_RPT_PAD, _CH), jnp.float32)

    degs = _deg_kernel()(ei0, ei1, ones16, zeros16)
    deg_in = degs[0]
    deg_out = degs[1]

    for l in range(_L):
        w_s = jnp.concatenate(
            [msg_W[l][:, :d].T, msg_rev_W[l][:, :d].T], axis=1)
        w_all = w_s.reshape(d, 2 * _NCH, _CH).transpose(1, 0, 2)
        tabs = _proj_tables(h, w_all)

        s_chunks = _scat_kernel()(tabs, ei0, ei1, zeros_ch)

        w1d = msg_W[l][:, d:].T.reshape(d, _NCH, _CH).transpose(1, 0, 2)
        w2d = msg_rev_W[l][:, d:].T.reshape(d, _NCH, _CH).transpose(1, 0, 2)
        b1c = msg_b[l].reshape(_NCH, _CH)
        b2c = msg_rev_b[l].reshape(_NCH, _CH)
        wih_t = W_ih[l].T.reshape(_NCH, _CH, 3 * d)
        whh_t = W_hh[l].T
        h = _gru_update(s_chunks, h, deg_in, deg_out, w1d, w2d, b1c, b2c,
                        wih_t, b_ih[l].reshape(1, 3 * d), whh_t,
                        b_hh[l].reshape(1, 3 * d))
    return h


# speedup vs baseline: 6.8180x; 1.7604x over previous
import functools

import jax
import jax.numpy as jnp
from jax import lax
from jax.experimental import pallas as pl
from jax.experimental.pallas import tpu as pltpu
from jax.experimental.pallas import tpu_sc as plsc

_N = 10000
_E = 160000
_D = 256
_L = 3
_CH = 128
_NCH = 2 * _D // _CH
_NS = 16
_NC = 2
_EPT = _E // _NS
_K = 80
_ITERS = _EPT // _K
_RPT_PAD = 640
_FPT = 624
_ROWBLK = 256
_GRID_R = (_N + _ROWBLK - 1) // _ROWBLK


def _deg_body(ei0_hbm, ei1_hbm, ones_hbm, zeros_hbm, out_hbm,
              idx_v, ones_v, dacc):
    c = lax.axis_index("c")
    s = lax.axis_index("s")
    pltpu.sync_copy(ones_hbm, ones_v)
    for core in range(_NC):
        @pl.when(c == core)
        def _(core=core):
            src = ei1_hbm if core == 0 else ei0_hbm
            pltpu.sync_copy(zeros_hbm, dacc.at[pl.ds(s * _RPT_PAD, _RPT_PAD)])
            plsc.subcore_barrier()

            def body(i, carry):
                base = pl.multiple_of(s * _EPT + i * _K, 8)
                pltpu.sync_copy(src.at[pl.ds(base, _K)], idx_v)
                pltpu.sync_copy(ones_v, dacc.at[idx_v], add=True)
                return carry

            lax.fori_loop(0, _ITERS, body, 0)
            plsc.subcore_barrier()
            pltpu.sync_copy(dacc.at[pl.ds(s * _FPT, _FPT)],
                            out_hbm.at[core].at[pl.ds(s * _FPT, _FPT)])

            @pl.when(s == _NS - 1)
            def _():
                tail = _NS * _FPT
                pltpu.sync_copy(dacc.at[pl.ds(tail, _N - tail)],
                                out_hbm.at[core].at[pl.ds(tail, _N - tail)])


@functools.cache
def _deg_kernel():
    return pl.kernel(
        _deg_body,
        out_type=jax.ShapeDtypeStruct((2, _N, _CH), jnp.float32),
        mesh=plsc.VectorSubcoreMesh(core_axis_name="c", subcore_axis_name="s",
                                    num_cores=_NC, num_subcores=_NS),
        scratch_types=[
            pltpu.VMEM((_K,), jnp.int32),
            pltpu.VMEM((_K, _CH), jnp.float32),
            pltpu.VMEM_SHARED((_NS * _RPT_PAD, _CH), jnp.float32),
        ],
    )


def _scat_body(tabs_hbm, ei0_hbm, ei1_hbm, zeros_hbm, out_hbm,
               i0a, i1a, i0b, i1b, ra0, ra1, rb0, rb1, acc,
               sia, sib, sa0, sa1, sb0, sb1):
    c = lax.axis_index("c")
    s = lax.axis_index("s")
    drows = tabs_hbm.at[0, pl.ds(0, _K)]
    didx = ei0_hbm.at[pl.ds(0, _K)]
    nhalf = (_ITERS - 1) // 2

    def load_idx(i, idx0_buf, idx1_buf, sem):
        base = pl.multiple_of(s * _EPT + i * _K, 8)
        pltpu.async_copy(ei0_hbm.at[pl.ds(base, _K)], idx0_buf, sem)
        pltpu.async_copy(ei1_hbm.at[pl.ds(base, _K)], idx1_buf, sem)

    def wait_idx(idx0_buf, idx1_buf, sem):
        pltpu.make_async_copy(didx, idx0_buf, sem).wait()
        pltpu.make_async_copy(didx, idx1_buf, sem).wait()

    for chunk in range(_NCH):
        @pl.when(c == chunk // (_NCH // _NC))
        def _(chunk=chunk):
            fwd = tabs_hbm.at[chunk]
            rev = tabs_hbm.at[chunk + _NCH]
            pltpu.sync_copy(zeros_hbm, acc.at[pl.ds(s * _RPT_PAD, _RPT_PAD)])
            plsc.subcore_barrier()
            load_idx(0, i0a, i1a, sia)
            wait_idx(i0a, i1a, sia)
            pltpu.async_copy(fwd.at[i0a], ra0, sa0)
            pltpu.async_copy(rev.at[i1a], rb0, sb0)
            load_idx(1, i0b, i1b, sib)

            def body(j, carry):
                pltpu.make_async_copy(drows, ra0, sa0).wait()
                pltpu.sync_copy(ra0, acc.at[i1a], add=True)
                pltpu.make_async_copy(drows, rb0, sb0).wait()
                pltpu.sync_copy(rb0, acc.at[i0a], add=True)
                load_idx(2 * j + 2, i0a, i1a, sia)
                wait_idx(i0b, i1b, sib)
                pltpu.async_copy(fwd.at[i0b], ra1, sa1)
                pltpu.async_copy(rev.at[i1b], rb1, sb1)
                pltpu.make_async_copy(drows, ra1, sa1).wait()
                pltpu.sync_copy(ra1, acc.at[i1b], add=True)
                pltpu.make_async_copy(drows, rb1, sb1).wait()
                pltpu.sync_copy(rb1, acc.at[i0b], add=True)

                @pl.when(j < nhalf - 1)
                def _():
                    load_idx(2 * j + 3, i0b, i1b, sib)

                wait_idx(i0a, i1a, sia)
                pltpu.async_copy(fwd.at[i0a], ra0, sa0)
                pltpu.async_copy(rev.at[i1a], rb0, sb0)
                return carry

            lax.fori_loop(0, nhalf, body, 0)
            pltpu.make_async_copy(drows, ra0, sa0).wait()
            pltpu.sync_copy(ra0, acc.at[i1a], add=True)
            pltpu.make_async_copy(drows, rb0, sb0).wait()
            pltpu.sync_copy(rb0, acc.at[i0a], add=True)
            plsc.subcore_barrier()
            pltpu.sync_copy(acc.at[pl.ds(s * _FPT, _FPT)],
                            out_hbm.at[chunk].at[pl.ds(s * _FPT, _FPT)])

            @pl.when(s == _NS - 1)
            def _():
                tail = _NS * _FPT
                pltpu.sync_copy(acc.at[pl.ds(tail, _N - tail)],
                                out_hbm.at[chunk].at[pl.ds(tail, _N - tail)])

            plsc.subcore_barrier()


@functools.cache
def _scat_kernel():
    return pl.kernel(
        _scat_body,
        out_type=jax.ShapeDtypeStruct((_NCH, _N, _CH), jnp.float32),
        mesh=plsc.VectorSubcoreMesh(core_axis_name="c", subcore_axis_name="s",
                                    num_cores=_NC, num_subcores=_NS),
        scratch_types=[
            pltpu.VMEM((_K,), jnp.int32),
            pltpu.VMEM((_K,), jnp.int32),
            pltpu.VMEM((_K,), jnp.int32),
            pltpu.VMEM((_K,), jnp.int32),
            pltpu.VMEM((_K, _CH), jnp.float32),
            pltpu.VMEM((_K, _CH), jnp.float32),
            pltpu.VMEM((_K, _CH), jnp.float32),
            pltpu.VMEM((_K, _CH), jnp.float32),
            pltpu.VMEM_SHARED((_NS * _RPT_PAD, _CH), jnp.float32),
            pltpu.SemaphoreType.DMA,
            pltpu.SemaphoreType.DMA,
            pltpu.SemaphoreType.DMA,
            pltpu.SemaphoreType.DMA,
            pltpu.SemaphoreType.DMA,
            pltpu.SemaphoreType.DMA,
        ],
    )


def _proj_body(h_ref, w_ref, out_ref):
    full = jnp.dot(h_ref[...], w_ref[...], preferred_element_type=jnp.float32)
    for c2 in range(2 * _NCH):
        out_ref[c2] = full[:, c2 * _CH:(c2 + 1) * _CH]


def _proj_tables(h, w_all):
    return pl.pallas_call(
        _proj_body,
        grid=(_GRID_R,),
        in_specs=[
            pl.BlockSpec((_ROWBLK, _D), lambda i: (i, 0)),
            pl.BlockSpec((_D, 4 * _D), lambda i: (0, 0)),
        ],
        out_specs=pl.BlockSpec((2 * _NCH, _ROWBLK, _CH), lambda i: (0, i, 0)),
        out_shape=jax.ShapeDtypeStruct((2 * _NCH, _N, _CH), jnp.float32),
    )(h, w_all)


def _upd_body(s_ref, h_ref, di_ref, do_ref, w1d_ref, w2d_ref, b1_ref, b2_ref,
              wih_ref, bih_ref, whh_ref, bhh_ref, out_ref):
    f32 = jnp.float32
    h_blk = h_ref[...]
    di = di_ref[:, 0:1]
    do = do_ref[:, 0:1]
    s_comb = jnp.concatenate([s_ref[c2] for c2 in range(_NCH)], axis=1)
    p1d = jnp.dot(h_blk, w1d_ref[...], preferred_element_type=f32)
    p2d = jnp.dot(h_blk, w2d_ref[...], preferred_element_type=f32)
    aggr = s_comb + di * (p1d + b1_ref[0]) + do * (p2d + b2_ref[0])
    gi = jnp.dot(aggr, wih_ref[...], preferred_element_type=f32) + bih_ref[0]
    gh = jnp.dot(h_blk, whh_ref[...], preferred_element_type=f32) + bhh_ref[0]
    r = jax.nn.sigmoid(gi[:, :_D] + gh[:, :_D])
    z = jax.nn.sigmoid(gi[:, _D:2 * _D] + gh[:, _D:2 * _D])
    ng = jnp.tanh(gi[:, 2 * _D:] + r * gh[:, 2 * _D:])
    out_ref[...] = (1.0 - z) * ng + z * h_blk


def _gru_update(s_chunks, h, deg_in, deg_out, w1d, w2d, b1, b2,
                wih_t, bih, whh_t, bhh):
    full = lambda shape: pl.BlockSpec(shape, lambda i: (0,) * len(shape))
    return pl.pallas_call(
        _upd_body,
        grid=(_GRID_R,),
        in_specs=[
            pl.BlockSpec((_NCH, _ROWBLK, _CH), lambda i: (0, i, 0)),
            pl.BlockSpec((_ROWBLK, _D), lambda i: (i, 0)),
            pl.BlockSpec((_ROWBLK, _CH), lambda i: (i, 0)),
            pl.BlockSpec((_ROWBLK, _CH), lambda i: (i, 0)),
            full((_D, 2 * _D)),
            full((_D, 2 * _D)),
            full((1, 2 * _D)),
            full((1, 2 * _D)),
            full((2 * _D, 3 * _D)),
            full((1, 3 * _D)),
            full((_D, 3 * _D)),
            full((1, 3 * _D)),
        ],
        out_specs=pl.BlockSpec((_ROWBLK, _D), lambda i: (i, 0)),
        out_shape=jax.ShapeDtypeStruct((_N, _D), jnp.float32),
    )(s_chunks, h, deg_in, deg_out, w1d, w2d, b1, b2,
      wih_t, bih, whh_t, bhh)


def kernel(h, edge_index, msg_W, msg_b, msg_rev_W, msg_rev_b,
           W_ih, b_ih, W_hh, b_hh):
    d = _D
    ei0 = edge_index[0].astype(jnp.int32)
    ei1 = edge_index[1].astype(jnp.int32)

    zeros_ch = jnp.zeros((_RPT_PAD, _CH), jnp.float32)
    ones_ch = jnp.ones((_K, _CH), jnp.float32)

    degs = _deg_kernel()(ei0, ei1, ones_ch, zeros_ch)
    deg_in = degs[0]
    deg_out = degs[1]

    for l in range(_L):
        w_all = jnp.concatenate(
            [msg_W[l][:, :d].T, msg_rev_W[l][:, :d].T], axis=1)
        tabs = _proj_tables(h, w_all)

        s_chunks = _scat_kernel()(tabs, ei0, ei1, zeros_ch)

        w1d = msg_W[l][:, d:].T
        w2d = msg_rev_W[l][:, d:].T
        h = _gru_update(s_chunks, h, deg_in, deg_out, w1d, w2d,
                        msg_b[l].reshape(1, 2 * d),
                        msg_rev_b[l].reshape(1, 2 * d),
                        W_ih[l].T, b_ih[l].reshape(1, 3 * d), W_hh[l].T,
                        b_hh[l].reshape(1, 3 * d))
    return h


# speedup vs baseline: 8.8385x; 1.2963x over previous
import functools

import jax
import jax.numpy as jnp
from jax import lax
from jax.experimental import pallas as pl
from jax.experimental.pallas import tpu as pltpu
from jax.experimental.pallas import tpu_sc as plsc

_N = 10000
_E = 160000
_D = 256
_L = 3
_CH = 128
_NCH = 2 * _D // _CH
_NS = 16
_NC = 2
_EPT = _E // _NS
_K = 80
_ITERS = _EPT // _K
_RPT_PAD = 640
_FPT = 624
_ROWBLK = 256
_GRID_R = (_N + _ROWBLK - 1) // _ROWBLK


def _deg_body(ei0_hbm, ei1_hbm, ones_hbm, zeros_hbm, out_hbm,
              idx_v, ones_v, dacc):
    c = lax.axis_index("c")
    s = lax.axis_index("s")
    pltpu.sync_copy(ones_hbm, ones_v)
    for core in range(_NC):
        @pl.when(c == core)
        def _(core=core):
            src = ei1_hbm if core == 0 else ei0_hbm
            pltpu.sync_copy(zeros_hbm, dacc.at[pl.ds(s * _RPT_PAD, _RPT_PAD)])
            plsc.subcore_barrier()

            def body(i, carry):
                base = pl.multiple_of(s * _EPT + i * _K, 8)
                pltpu.sync_copy(src.at[pl.ds(base, _K)], idx_v)
                pltpu.sync_copy(ones_v, dacc.at[idx_v], add=True)
                return carry

            lax.fori_loop(0, _ITERS, body, 0)
            plsc.subcore_barrier()
            pltpu.sync_copy(dacc.at[pl.ds(s * _FPT, _FPT)],
                            out_hbm.at[core].at[pl.ds(s * _FPT, _FPT)])

            @pl.when(s == _NS - 1)
            def _():
                tail = _NS * _FPT
                pltpu.sync_copy(dacc.at[pl.ds(tail, _N - tail)],
                                out_hbm.at[core].at[pl.ds(tail, _N - tail)])


@functools.cache
def _deg_kernel():
    return pl.kernel(
        _deg_body,
        out_type=jax.ShapeDtypeStruct((2, _N, _CH), jnp.float32),
        mesh=plsc.VectorSubcoreMesh(core_axis_name="c", subcore_axis_name="s",
                                    num_cores=_NC, num_subcores=_NS),
        scratch_types=[
            pltpu.VMEM((_K,), jnp.int32),
            pltpu.VMEM((_K, _CH), jnp.float32),
            pltpu.VMEM_SHARED((_NS * _RPT_PAD, _CH), jnp.float32),
        ],
    )


def _scat_body(tabs_hbm, ei0_hbm, ei1_hbm, zeros_hbm, out_hbm,
               i0a, i1a, i0b, i1b, ra0, ra1, rb0, rb1, acc,
               sia, sib, sa0, sa1, sb0, sb1, ssa, ssb):
    c = lax.axis_index("c")
    s = lax.axis_index("s")
    drows = tabs_hbm.at[0, pl.ds(0, _K)]
    didx = ei0_hbm.at[pl.ds(0, _K)]
    nhalf = (_ITERS - 1) // 2

    def load_idx(i, idx0_buf, idx1_buf, sem):
        base = pl.multiple_of(s * _EPT + i * _K, 8)
        pltpu.async_copy(ei0_hbm.at[pl.ds(base, _K)], idx0_buf, sem)
        pltpu.async_copy(ei1_hbm.at[pl.ds(base, _K)], idx1_buf, sem)

    def wait_idx(idx0_buf, idx1_buf, sem):
        pltpu.make_async_copy(didx, idx0_buf, sem).wait()
        pltpu.make_async_copy(didx, idx1_buf, sem).wait()

    for chunk in range(_NCH):
        @pl.when(c == chunk // (_NCH // _NC))
        def _(chunk=chunk):
            fwd = tabs_hbm.at[chunk]
            rev = tabs_hbm.at[chunk + _NCH]
            pltpu.sync_copy(zeros_hbm, acc.at[pl.ds(s * _RPT_PAD, _RPT_PAD)])
            plsc.subcore_barrier()
            load_idx(0, i0a, i1a, sia)
            wait_idx(i0a, i1a, sia)
            pltpu.async_copy(fwd.at[i0a], ra0, sa0)
            pltpu.async_copy(rev.at[i1a], rb0, sb0)
            load_idx(1, i0b, i1b, sib)

            def body(j, carry):
                pltpu.make_async_copy(drows, ra0, sa0).wait()
                pltpu.async_copy(ra0, acc.at[i1a], ssa, add=True)
                pltpu.make_async_copy(drows, rb0, sb0).wait()
                pltpu.async_copy(rb0, acc.at[i0a], ssa, add=True)
                wait_idx(i0b, i1b, sib)
                pltpu.async_copy(fwd.at[i0b], ra1, sa1)
                pltpu.async_copy(rev.at[i1b], rb1, sb1)
                pltpu.make_async_copy(drows, ra0, ssa).wait()
                pltpu.make_async_copy(drows, rb0, ssa).wait()
                load_idx(2 * j + 2, i0a, i1a, sia)
                pltpu.make_async_copy(drows, ra1, sa1).wait()
                pltpu.async_copy(ra1, acc.at[i1b], ssb, add=True)
                pltpu.make_async_copy(drows, rb1, sb1).wait()
                pltpu.async_copy(rb1, acc.at[i0b], ssb, add=True)
                wait_idx(i0a, i1a, sia)
                pltpu.async_copy(fwd.at[i0a], ra0, sa0)
                pltpu.async_copy(rev.at[i1a], rb0, sb0)
                pltpu.make_async_copy(drows, ra1, ssb).wait()
                pltpu.make_async_copy(drows, rb1, ssb).wait()

                @pl.when(j < nhalf - 1)
                def _():
                    load_idx(2 * j + 3, i0b, i1b, sib)

                return carry

            lax.fori_loop(0, nhalf, body, 0)
            pltpu.make_async_copy(drows, ra0, sa0).wait()
            pltpu.sync_copy(ra0, acc.at[i1a], add=True)
            pltpu.make_async_copy(drows, rb0, sb0).wait()
            pltpu.sync_copy(rb0, acc.at[i0a], add=True)
            plsc.subcore_barrier()
            pltpu.sync_copy(acc.at[pl.ds(s * _FPT, _FPT)],
                            out_hbm.at[chunk].at[pl.ds(s * _FPT, _FPT)])

            @pl.when(s == _NS - 1)
            def _():
                tail = _NS * _FPT
                pltpu.sync_copy(acc.at[pl.ds(tail, _N - tail)],
                                out_hbm.at[chunk].at[pl.ds(tail, _N - tail)])

            plsc.subcore_barrier()


@functools.cache
def _scat_kernel():
    return pl.kernel(
        _scat_body,
        out_type=jax.ShapeDtypeStruct((_NCH, _N, _CH), jnp.float32),
        mesh=plsc.VectorSubcoreMesh(core_axis_name="c", subcore_axis_name="s",
                                    num_cores=_NC, num_subcores=_NS),
        scratch_types=[
            pltpu.VMEM((_K,), jnp.int32),
            pltpu.VMEM((_K,), jnp.int32),
            pltpu.VMEM((_K,), jnp.int32),
            pltpu.VMEM((_K,), jnp.int32),
            pltpu.VMEM((_K, _CH), jnp.float32),
            pltpu.VMEM((_K, _CH), jnp.float32),
            pltpu.VMEM((_K, _CH), jnp.float32),
            pltpu.VMEM((_K, _CH), jnp.float32),
            pltpu.VMEM_SHARED((_NS * _RPT_PAD, _CH), jnp.float32),
            pltpu.SemaphoreType.DMA,
            pltpu.SemaphoreType.DMA,
            pltpu.SemaphoreType.DMA,
            pltpu.SemaphoreType.DMA,
            pltpu.SemaphoreType.DMA,
            pltpu.SemaphoreType.DMA,
            pltpu.SemaphoreType.DMA,
            pltpu.SemaphoreType.DMA,
        ],
    )


def _proj_body(h_ref, w_ref, out_ref):
    full = jnp.dot(h_ref[...], w_ref[...], preferred_element_type=jnp.float32)
    for c2 in range(2 * _NCH):
        out_ref[c2] = full[:, c2 * _CH:(c2 + 1) * _CH]


def _proj_tables(h, w_all):
    return pl.pallas_call(
        _proj_body,
        grid=(_GRID_R,),
        in_specs=[
            pl.BlockSpec((_ROWBLK, _D), lambda i: (i, 0)),
            pl.BlockSpec((_D, 4 * _D), lambda i: (0, 0)),
        ],
        out_specs=pl.BlockSpec((2 * _NCH, _ROWBLK, _CH), lambda i: (0, i, 0)),
        out_shape=jax.ShapeDtypeStruct((2 * _NCH, _N, _CH), jnp.float32),
    )(h, w_all)


def _upd_body(s_ref, h_ref, di_ref, do_ref, w1d_ref, w2d_ref, b1_ref, b2_ref,
              wih_ref, bih_ref, whh_ref, bhh_ref, out_ref):
    f32 = jnp.float32
    h_blk = h_ref[...]
    di = di_ref[:, 0:1]
    do = do_ref[:, 0:1]
    s_comb = jnp.concatenate([s_ref[c2] for c2 in range(_NCH)], axis=1)
    p1d = jnp.dot(h_blk, w1d_ref[...], preferred_element_type=f32)
    p2d = jnp.dot(h_blk, w2d_ref[...], preferred_element_type=f32)
    aggr = s_comb + di * (p1d + b1_ref[0]) + do * (p2d + b2_ref[0])
    gi = jnp.dot(aggr, wih_ref[...], preferred_element_type=f32) + bih_ref[0]
    gh = jnp.dot(h_blk, whh_ref[...], preferred_element_type=f32) + bhh_ref[0]
    r = jax.nn.sigmoid(gi[:, :_D] + gh[:, :_D])
    z = jax.nn.sigmoid(gi[:, _D:2 * _D] + gh[:, _D:2 * _D])
    ng = jnp.tanh(gi[:, 2 * _D:] + r * gh[:, 2 * _D:])
    out_ref[...] = (1.0 - z) * ng + z * h_blk


def _gru_update(s_chunks, h, deg_in, deg_out, w1d, w2d, b1, b2,
                wih_t, bih, whh_t, bhh):
    full = lambda shape: pl.BlockSpec(shape, lambda i: (0,) * len(shape))
    return pl.pallas_call(
        _upd_body,
        grid=(_GRID_R,),
        in_specs=[
            pl.BlockSpec((_NCH, _ROWBLK, _CH), lambda i: (0, i, 0)),
            pl.BlockSpec((_ROWBLK, _D), lambda i: (i, 0)),
            pl.BlockSpec((_ROWBLK, _CH), lambda i: (i, 0)),
            pl.BlockSpec((_ROWBLK, _CH), lambda i: (i, 0)),
            full((_D, 2 * _D)),
            full((_D, 2 * _D)),
            full((1, 2 * _D)),
            full((1, 2 * _D)),
            full((2 * _D, 3 * _D)),
            full((1, 3 * _D)),
            full((_D, 3 * _D)),
            full((1, 3 * _D)),
        ],
        out_specs=pl.BlockSpec((_ROWBLK, _D), lambda i: (i, 0)),
        out_shape=jax.ShapeDtypeStruct((_N, _D), jnp.float32),
    )(s_chunks, h, deg_in, deg_out, w1d, w2d, b1, b2,
      wih_t, bih, whh_t, bhh)


def kernel(h, edge_index, msg_W, msg_b, msg_rev_W, msg_rev_b,
           W_ih, b_ih, W_hh, b_hh):
    d = _D
    ei0 = edge_index[0].astype(jnp.int32)
    ei1 = edge_index[1].astype(jnp.int32)

    zeros_ch = jnp.zeros((_RPT_PAD, _CH), jnp.float32)
    ones_ch = jnp.ones((_K, _CH), jnp.float32)

    degs = _deg_kernel()(ei0, ei1, ones_ch, zeros_ch)
    deg_in = degs[0]
    deg_out = degs[1]

    for l in range(_L):
        w_all = jnp.concatenate(
            [msg_W[l][:, :d].T, msg_rev_W[l][:, :d].T], axis=1)
        tabs = _proj_tables(h, w_all)

        s_chunks = _scat_kernel()(tabs, ei0, ei1, zeros_ch)

        w1d = msg_W[l][:, d:].T
        w2d = msg_rev_W[l][:, d:].T
        h = _gru_update(s_chunks, h, deg_in, deg_out, w1d, w2d,
                        msg_b[l].reshape(1, 2 * d),
                        msg_rev_b[l].reshape(1, 2 * d),
                        W_ih[l].T, b_ih[l].reshape(1, 3 * d), W_hh[l].T,
                        b_hh[l].reshape(1, 3 * d))
    return h
